# initial kernel scaffold (unmeasured)
import jax
import jax.numpy as jnp
from jax import lax
from jax.experimental import pallas as pl
from jax.experimental.pallas import tpu as pltpu

N_DEV = 4
N_LAYERS = 3
B = 64
D = 2048
H = 4096
BLK = 512
N_WIN_BLK = D // BLK
N_WOUT_BLK = H // BLK


def kernel(x, Win0, Wout0, Win1, Wout1, Win2, Wout2):
    def body(x_ref, win0, wout0, win1, wout1, win2, wout2, out_ref,
             win_buf, wout_buf, send_buf, recv_buf,
             win_sems, wout_sems, send_sems, recv_sems):
        my = lax.axis_index("i")
        wins = [win0, win1, win2]
        wouts = [wout0, wout1, wout2]

        barrier = pltpu.get_barrier_semaphore()
        for t in range(1, N_DEV):
            pl.semaphore_signal(
                barrier, inc=1,
                device_id=((my + t) % N_DEV,),
                device_id_type=pl.DeviceIdType.MESH,
            )
        pl.semaphore_wait(barrier, N_DEV - 1)

        def win_copy(layer, kb):
            return pltpu.make_async_copy(
                wins[layer].at[pl.ds(kb * BLK, BLK), :],
                win_buf.at[kb % 2],
                win_sems.at[kb % 2],
            )

        def wout_copy(layer, jb):
            return pltpu.make_async_copy(
                wouts[layer].at[pl.ds(jb * BLK, BLK), :],
                wout_buf.at[jb % 2],
                wout_sems.at[jb % 2],
            )

        def ar_rdma(layer, t):
            return pltpu.make_async_remote_copy(
                src_ref=send_buf,
                dst_ref=recv_buf.at[layer, t - 1],
                send_sem=send_sems.at[t - 1],
                recv_sem=recv_sems.at[layer, t - 1],
                device_id=((my + t) % N_DEV,),
                device_id_type=pl.DeviceIdType.MESH,
            )

        win_copy(0, 0).start()
        win_copy(0, 1).start()

        x_bf = x_ref[:, :].astype(jnp.bfloat16)

        for layer in range(N_LAYERS):
            h_acc = jnp.zeros((B, H), jnp.float32)
            for kb in range(N_WIN_BLK):
                win_copy(layer, kb).wait()
                w = win_buf[kb % 2].astype(jnp.bfloat16)
                h_acc = h_acc + jnp.dot(
                    x_bf[:, kb * BLK:(kb + 1) * BLK], w,
                    preferred_element_type=jnp.float32,
                )
                if kb + 2 < N_WIN_BLK:
                    win_copy(layer, kb + 2).start()
                elif kb + 2 == N_WIN_BLK:
                    wout_copy(layer, 0).start()
                else:
                    wout_copy(layer, 1).start()
            h_bf = jnp.maximum(h_acc, 0.0).astype(jnp.bfloat16)

            p_acc = jnp.zeros((B, D), jnp.float32)
            for jb in range(N_WOUT_BLK):
                wout_copy(layer, jb).wait()
                w = wout_buf[jb % 2].astype(jnp.bfloat16)
                p_acc = p_acc + jnp.dot(
                    h_bf[:, jb * BLK:(jb + 1) * BLK], w,
                    preferred_element_type=jnp.float32,
                )
                if jb + 2 < N_WOUT_BLK:
                    wout_copy(layer, jb + 2).start()

            send_buf[:, :] = p_acc.astype(jnp.bfloat16)
            for t in range(1, N_DEV):
                ar_rdma(layer, t).start()
            if layer + 1 < N_LAYERS:
                win_copy(layer + 1, 0).start()
                win_copy(layer + 1, 1).start()
            acc = p_acc
            for t in range(1, N_DEV):
                ar_rdma(layer, t).wait_recv()
                acc = acc + recv_buf[layer, t - 1].astype(jnp.float32)
            for t in range(1, N_DEV):
                ar_rdma(layer, t).wait_send()
            if layer + 1 < N_LAYERS:
                x_bf = acc.astype(jnp.bfloat16)
            else:
                out_ref[:, :] = acc

    return pl.pallas_call(
        body,
        out_shape=jax.ShapeDtypeStruct((B, D), jnp.float32),
        in_specs=[
            pl.BlockSpec(memory_space=pltpu.VMEM),
            pl.BlockSpec(memory_space=pltpu.ANY),
            pl.BlockSpec(memory_space=pltpu.ANY),
            pl.BlockSpec(memory_space=pltpu.ANY),
            pl.BlockSpec(memory_space=pltpu.ANY),
            pl.BlockSpec(memory_space=pltpu.ANY),
            pl.BlockSpec(memory_space=pltpu.ANY),
        ],
        out_specs=pl.BlockSpec(memory_space=pltpu.VMEM),
        scratch_shapes=[
            pltpu.VMEM((2, BLK, H), jnp.float32),
            pltpu.VMEM((2, BLK, D), jnp.float32),
            pltpu.VMEM((B, D), jnp.bfloat16),
            pltpu.VMEM((N_LAYERS, 3, B, D), jnp.bfloat16),
            pltpu.SemaphoreType.DMA((2,)),
            pltpu.SemaphoreType.DMA((2,)),
            pltpu.SemaphoreType.DMA((3,)),
            pltpu.SemaphoreType.DMA((N_LAYERS, 3)),
        ],
        compiler_params=pltpu.CompilerParams(collective_id=0),
    )(x, Win0, Wout0, Win1, Wout1, Win2, Wout2)


# baseline (device time: 91281 ns/iter reference)
import jax
import jax.numpy as jnp
from jax import lax
from jax.experimental import pallas as pl
from jax.experimental.pallas import tpu as pltpu

N_DEV = 4
N_LAYERS = 3
B = 64
D = 2048
H = 4096
BLK = 512
N_WIN_BLK = D // BLK
N_WOUT_BLK = H // BLK


def kernel(x, Win0, Wout0, Win1, Wout1, Win2, Wout2):
    def body(x_ref, win0, wout0, win1, wout1, win2, wout2, out_ref,
             win_buf, wout_buf, send_buf, recv_buf,
             win_sems, wout_sems, send_sems, recv_sems):
        my = lax.axis_index("i")
        wins = [win0, win1, win2]
        wouts = [wout0, wout1, wout2]

        barrier = pltpu.get_barrier_semaphore()
        for t in range(1, N_DEV):
            pl.semaphore_signal(
                barrier, inc=1,
                device_id=((my + t) % N_DEV,),
                device_id_type=pl.DeviceIdType.MESH,
            )
        pl.semaphore_wait(barrier, N_DEV - 1)

        def win_copy(layer, kb):
            return pltpu.make_async_copy(
                wins[layer].at[pl.ds(kb * BLK, BLK), :],
                win_buf.at[kb % 2],
                win_sems.at[kb % 2],
            )

        def wout_copy(layer, jb):
            return pltpu.make_async_copy(
                wouts[layer].at[pl.ds(jb * BLK, BLK), :],
                wout_buf.at[jb % 2],
                wout_sems.at[jb % 2],
            )

        def ar_rdma(layer, t):
            return pltpu.make_async_remote_copy(
                src_ref=send_buf,
                dst_ref=recv_buf.at[layer, t - 1],
                send_sem=send_sems.at[t - 1],
                recv_sem=recv_sems.at[layer, t - 1],
                device_id=((my + t) % N_DEV,),
                device_id_type=pl.DeviceIdType.MESH,
            )

        win_copy(0, 0).start()
        win_copy(0, 1).start()

        x_bf = x_ref[:, :].astype(jnp.bfloat16)

        for layer in range(N_LAYERS):
            h_acc = jnp.zeros((B, H), jnp.float32)
            for kb in range(N_WIN_BLK):
                win_copy(layer, kb).wait()
                w = win_buf[kb % 2].astype(jnp.bfloat16)
                h_acc = h_acc + jnp.dot(
                    x_bf[:, kb * BLK:(kb + 1) * BLK], w,
                    preferred_element_type=jnp.float32,
                )
                if kb + 2 < N_WIN_BLK:
                    win_copy(layer, kb + 2).start()
                elif kb + 2 == N_WIN_BLK:
                    wout_copy(layer, 0).start()
                else:
                    wout_copy(layer, 1).start()
            h_bf = jnp.maximum(h_acc, 0.0).astype(jnp.bfloat16)

            p_acc = jnp.zeros((B, D), jnp.float32)
            for jb in range(N_WOUT_BLK):
                wout_copy(layer, jb).wait()
                w = wout_buf[jb % 2].astype(jnp.bfloat16)
                p_acc = p_acc + jnp.dot(
                    h_bf[:, jb * BLK:(jb + 1) * BLK], w,
                    preferred_element_type=jnp.float32,
                )
                if jb + 2 < N_WOUT_BLK:
                    wout_copy(layer, jb + 2).start()

            send_buf[:, :] = p_acc.astype(jnp.bfloat16)
            for t in range(1, N_DEV):
                ar_rdma(layer, t).start()
            if layer + 1 < N_LAYERS:
                win_copy(layer + 1, 0).start()
                win_copy(layer + 1, 1).start()
            acc = p_acc
            for t in range(1, N_DEV):
                ar_rdma(layer, t).wait_recv()
                acc = acc + recv_buf[layer, t - 1].astype(jnp.float32)
            for t in range(1, N_DEV):
                ar_rdma(layer, t).wait_send()
            if layer + 1 < N_LAYERS:
                x_bf = acc.astype(jnp.bfloat16)
            else:
                out_ref[:, :] = acc

    return pl.pallas_call(
        body,
        out_shape=jax.ShapeDtypeStruct((B, D), jnp.float32),
        in_specs=[
            pl.BlockSpec(memory_space=pltpu.VMEM),
            pl.BlockSpec(memory_space=pl.ANY),
            pl.BlockSpec(memory_space=pl.ANY),
            pl.BlockSpec(memory_space=pl.ANY),
            pl.BlockSpec(memory_space=pl.ANY),
            pl.BlockSpec(memory_space=pl.ANY),
            pl.BlockSpec(memory_space=pl.ANY),
        ],
        out_specs=pl.BlockSpec(memory_space=pltpu.VMEM),
        scratch_shapes=[
            pltpu.VMEM((2, BLK, H), jnp.float32),
            pltpu.VMEM((2, BLK, D), jnp.float32),
            pltpu.VMEM((B, D), jnp.bfloat16),
            pltpu.VMEM((N_LAYERS, 3, B, D), jnp.bfloat16),
            pltpu.SemaphoreType.DMA((2,)),
            pltpu.SemaphoreType.DMA((2,)),
            pltpu.SemaphoreType.DMA((3,)),
            pltpu.SemaphoreType.DMA((N_LAYERS, 3)),
        ],
        compiler_params=pltpu.CompilerParams(
            collective_id=0,
            vmem_limit_bytes=64 * 1024 * 1024,
        ),
    )(x, Win0, Wout0, Win1, Wout1, Win2, Wout2)


# device time: 80829 ns/iter; 1.1293x vs baseline; 1.1293x over previous
import jax
import jax.numpy as jnp
from jax import lax
from jax.experimental import pallas as pl
from jax.experimental.pallas import tpu as pltpu

N_DEV = 4
N_LAYERS = 3
B = 64
D = 2048
H = 4096
BLK = 512
N_WIN_BLK = D // BLK
N_WOUT_BLK = H // BLK
NSLOT = 4


def kernel(x, Win0, Wout0, Win1, Wout1, Win2, Wout2):
    def body(x_ref, win0, wout0, win1, wout1, win2, wout2, out_ref,
             win_buf, wout_buf, send_buf, recv_buf,
             win_sems, wout_sems, send_sems, recv_sems):
        my = lax.axis_index("i")
        wins = [win0, win1, win2]
        wouts = [wout0, wout1, wout2]

        barrier = pltpu.get_barrier_semaphore()
        for t in range(1, N_DEV):
            pl.semaphore_signal(
                barrier, inc=1,
                device_id=((my + t) % N_DEV,),
                device_id_type=pl.DeviceIdType.MESH,
            )
        pl.semaphore_wait(barrier, N_DEV - 1)

        def win_copy(layer, kb):
            return pltpu.make_async_copy(
                wins[layer].at[pl.ds(kb * BLK, BLK), :],
                win_buf.at[kb % NSLOT],
                win_sems.at[kb % NSLOT],
            )

        def wout_copy(layer, jb):
            return pltpu.make_async_copy(
                wouts[layer].at[pl.ds(jb * BLK, BLK), :],
                wout_buf.at[jb % NSLOT],
                wout_sems.at[jb % NSLOT],
            )

        def ar_rdma(layer, t):
            return pltpu.make_async_remote_copy(
                src_ref=send_buf,
                dst_ref=recv_buf.at[layer, t - 1],
                send_sem=send_sems.at[t - 1],
                recv_sem=recv_sems.at[layer, t - 1],
                device_id=((my + t) % N_DEV,),
                device_id_type=pl.DeviceIdType.MESH,
            )

        def prefetch_layer(layer):
            for kb in range(N_WIN_BLK):
                win_copy(layer, kb).start()
            wout_copy(layer, 0).start()
            wout_copy(layer, 1).start()

        prefetch_layer(0)
        x_f32 = x_ref[:, :]

        for layer in range(N_LAYERS):
            h_acc = jnp.zeros((B, H), jnp.float32)
            for kb in range(N_WIN_BLK):
                win_copy(layer, kb).wait()
                h_acc = h_acc + jnp.dot(
                    x_f32[:, kb * BLK:(kb + 1) * BLK], win_buf[kb % NSLOT],
                    preferred_element_type=jnp.float32,
                )
                if kb < 2:
                    wout_copy(layer, kb + 2).start()
            h_f32 = jnp.maximum(h_acc, 0.0)

            p_acc = jnp.zeros((B, D), jnp.float32)
            for jb in range(N_WOUT_BLK):
                wout_copy(layer, jb).wait()
                p_acc = p_acc + jnp.dot(
                    h_f32[:, jb * BLK:(jb + 1) * BLK], wout_buf[jb % NSLOT],
                    preferred_element_type=jnp.float32,
                )
                if jb + NSLOT < N_WOUT_BLK:
                    wout_copy(layer, jb + NSLOT).start()

            send_buf[:, :] = p_acc.astype(jnp.bfloat16)
            for t in range(1, N_DEV):
                ar_rdma(layer, t).start()
            if layer + 1 < N_LAYERS:
                prefetch_layer(layer + 1)
            acc = p_acc
            for t in range(1, N_DEV):
                ar_rdma(layer, t).wait_recv()
                acc = acc + recv_buf[layer, t - 1].astype(jnp.float32)
            for t in range(1, N_DEV):
                ar_rdma(layer, t).wait_send()
            if layer + 1 < N_LAYERS:
                x_f32 = acc
            else:
                out_ref[:, :] = acc

    return pl.pallas_call(
        body,
        out_shape=jax.ShapeDtypeStruct((B, D), jnp.float32),
        in_specs=[
            pl.BlockSpec(memory_space=pltpu.VMEM),
            pl.BlockSpec(memory_space=pl.ANY),
            pl.BlockSpec(memory_space=pl.ANY),
            pl.BlockSpec(memory_space=pl.ANY),
            pl.BlockSpec(memory_space=pl.ANY),
            pl.BlockSpec(memory_space=pl.ANY),
            pl.BlockSpec(memory_space=pl.ANY),
        ],
        out_specs=pl.BlockSpec(memory_space=pltpu.VMEM),
        scratch_shapes=[
            pltpu.VMEM((NSLOT, BLK, H), jnp.float32),
            pltpu.VMEM((NSLOT, BLK, D), jnp.float32),
            pltpu.VMEM((B, D), jnp.bfloat16),
            pltpu.VMEM((N_LAYERS, 3, B, D), jnp.bfloat16),
            pltpu.SemaphoreType.DMA((NSLOT,)),
            pltpu.SemaphoreType.DMA((NSLOT,)),
            pltpu.SemaphoreType.DMA((3,)),
            pltpu.SemaphoreType.DMA((N_LAYERS, 3)),
        ],
        compiler_params=pltpu.CompilerParams(
            collective_id=0,
            vmem_limit_bytes=100 * 1024 * 1024,
        ),
    )(x, Win0, Wout0, Win1, Wout1, Win2, Wout2)
